# Initial kernel scaffold; baseline (speedup 1.0000x reference)
import jax
import jax.numpy as jnp
from jax.experimental import pallas as pl


def kernel(src, W_emb, pos_emb):
    raise NotImplementedError("write your pallas kernel here")


# speedup vs baseline: 1.4765x; 1.4765x over previous
import functools

import jax
import jax.numpy as jnp
from jax import lax
from jax.experimental import pallas as pl
from jax.experimental.pallas import tpu as pltpu
from jax.experimental.pallas import tpu_sc as plsc

B = 32
L = 2160
HID = 768
NOUT = L // 9
C = 8
NCHUNK = NOUT // C
LANES = 16
NC = 2
SCALE = float(768.0 ** 0.5)

_mesh = plsc.VectorSubcoreMesh(core_axis_name="c", subcore_axis_name="s")


@functools.partial(
    pl.kernel,
    mesh=_mesh,
    out_type=[
        jax.ShapeDtypeStruct((B, NOUT, HID), jnp.float32),
        jax.ShapeDtypeStruct((B, NOUT), jnp.int32),
    ],
    scratch_types=[
        pltpu.VMEM((L,), jnp.int32),
        pltpu.VMEM((9 * C, HID), jnp.float32),
        pltpu.VMEM((9 * C, HID), jnp.float32),
        pltpu.VMEM((C, HID), jnp.float32),
        pltpu.VMEM((NOUT,), jnp.int32),
        pltpu.VMEM((L,), jnp.int32),
        pltpu.VMEM((L,), jnp.int32),
        pltpu.SemaphoreType.DMA,
        pltpu.SemaphoreType.DMA,
    ],
)
def _enc(src_hbm, w_hbm, pos_hbm, h_hbm, m_hbm,
         srow, wbuf, pbuf, obuf, mbuf, gidx, tbuf, sem_w, sem_p):
    b = lax.axis_index("s") * NC + lax.axis_index("c")
    pltpu.sync_copy(src_hbm.at[pl.ds(b * L, L)], srow)

    def fill(q, carry):
        off = q // (NOUT // LANES)
        base = 9 * LANES * q - (L - 1) * off + b * L
        gidx[pl.ds(LANES * q, LANES)] = lax.iota(jnp.int32, LANES) * 9 + base
        return carry

    lax.fori_loop(0, L // LANES, fill, 0)
    pltpu.async_copy(src_hbm.at[gidx], tbuf, sem_w).wait()

    for mblk in range(NOUT // LANES):
        def gsum(t):
            s = None
            for u in range(3):
                v = tbuf[pl.ds((3 * t + u) * NOUT + LANES * mblk, LANES)]
                s = v if s is None else s + v
            return s.astype(jnp.float32)

        mval = jnp.maximum(jnp.maximum(gsum(0), gsum(1)), gsum(2))
        mbuf[pl.ds(LANES * mblk, LANES)] = (
            jnp.where(mval != 0.0, 1, 0).astype(jnp.int32))
    pltpu.sync_copy(mbuf, m_hbm.at[b])

    def chunk(i, carry):
        idx = srow.at[pl.ds(9 * C * i, 9 * C)]
        cp_w = pltpu.async_copy(w_hbm.at[idx], wbuf, sem_w)
        cp_p = pltpu.async_copy(pos_hbm.at[pl.ds(9 * C * i, 9 * C)], pbuf, sem_p)
        cp_w.wait()
        cp_p.wait()
        for r in range(C):
            def col(vi, c2):
                sl = pl.ds(vi * LANES, LANES)
                best = None
                for t in range(3):
                    k = 9 * r + 3 * t
                    ws = wbuf[k, sl] + wbuf[k + 1, sl] + wbuf[k + 2, sl]
                    ps = pbuf[k, sl] + pbuf[k + 1, sl] + pbuf[k + 2, sl]
                    cand = ws * (SCALE / 3.0) + ps * (1.0 / 3.0)
                    best = cand if best is None else jnp.maximum(best, cand)
                obuf[r, sl] = best
                return c2
            lax.fori_loop(0, HID // LANES, col, 0)
        pltpu.sync_copy(obuf, h_hbm.at[b, pl.ds(C * i, C)])
        return carry

    lax.fori_loop(0, NCHUNK, chunk, 0)


def kernel(src, W_emb, pos_emb):
    h, m = _enc(src.reshape(-1), W_emb, pos_emb)
    return h, (m != 0)[:, None, None, :]


# speedup vs baseline: 2.1164x; 1.4334x over previous
import functools

import jax
import jax.numpy as jnp
from jax import lax
from jax.experimental import pallas as pl
from jax.experimental.pallas import tpu as pltpu
from jax.experimental.pallas import tpu_sc as plsc

B = 32
L = 2160
HID = 768
NOUT = L // 9
C = 8
NCHUNK = NOUT // C
LANES = 16
NC = 2
SCALE = float(768.0 ** 0.5)

_mesh = plsc.VectorSubcoreMesh(core_axis_name="c", subcore_axis_name="s")


@functools.partial(
    pl.kernel,
    mesh=_mesh,
    out_type=[
        jax.ShapeDtypeStruct((B, NOUT, HID), jnp.float32),
        jax.ShapeDtypeStruct((B, NOUT), jnp.int32),
    ],
    scratch_types=[
        pltpu.VMEM((L,), jnp.int32),
        pltpu.VMEM((9 * C, HID), jnp.float32),
        pltpu.VMEM((3 * C, HID), jnp.float32),
        pltpu.VMEM((C, HID), jnp.float32),
        pltpu.VMEM((NOUT,), jnp.int32),
        pltpu.VMEM((L,), jnp.int32),
        pltpu.VMEM((L,), jnp.int32),
        pltpu.VMEM_SHARED((L // 3, HID), jnp.float32),
        pltpu.SemaphoreType.DMA,
        pltpu.SemaphoreType.DMA,
    ],
)
def _enc(src_hbm, w_hbm, pos_hbm, h_hbm, m_hbm,
         srow, wbuf, pbuf, obuf, mbuf, gidx, tbuf, shared_pos, sem_w, sem_p):
    sid = lax.axis_index("s")
    b = sid * NC + lax.axis_index("c")
    pltpu.sync_copy(src_hbm.at[pl.ds(b * L, L)], srow)

    @pl.when(sid < 15)
    def _stage():
        for g in range(2):
            k0 = 48 * sid + 24 * g
            pltpu.sync_copy(pos_hbm.at[pl.ds(3 * k0, 9 * C)], wbuf)
            for r in range(3 * C):
                def pcol(vi, c2):
                    sl = pl.ds(vi * LANES, LANES)
                    pbuf[r, sl] = (
                        (wbuf[3 * r, sl] + wbuf[3 * r + 1, sl]
                         + wbuf[3 * r + 2, sl]) * (1.0 / SCALE))
                    return c2
                lax.fori_loop(0, HID // LANES, pcol, 0)
            pltpu.sync_copy(pbuf, shared_pos.at[pl.ds(k0, 3 * C)])
    plsc.subcore_barrier()

    def fill(q, carry):
        off = q // (NOUT // LANES)
        base = 9 * LANES * q - (L - 1) * off + b * L
        gidx[pl.ds(LANES * q, LANES)] = lax.iota(jnp.int32, LANES) * 9 + base
        return carry

    lax.fori_loop(0, L // LANES, fill, 0)
    pltpu.async_copy(src_hbm.at[gidx], tbuf, sem_w).wait()

    for mblk in range(NOUT // LANES):
        def gsum(t):
            s = None
            for u in range(3):
                v = tbuf[pl.ds((3 * t + u) * NOUT + LANES * mblk, LANES)]
                s = v if s is None else s + v
            return s.astype(jnp.float32)

        mval = jnp.maximum(jnp.maximum(gsum(0), gsum(1)), gsum(2))
        mbuf[pl.ds(LANES * mblk, LANES)] = (
            jnp.where(mval != 0.0, 1, 0).astype(jnp.int32))
    pltpu.sync_copy(mbuf, m_hbm.at[b])

    def chunk(i, carry):
        idx = srow.at[pl.ds(9 * C * i, 9 * C)]
        cp_w = pltpu.async_copy(w_hbm.at[idx], wbuf, sem_w)
        cp_p = pltpu.async_copy(shared_pos.at[pl.ds(3 * C * i, 3 * C)], pbuf, sem_p)
        cp_w.wait()
        cp_p.wait()
        for r in range(C):
            def col(vi, c2):
                sl = pl.ds(vi * LANES, LANES)
                best = None
                for t in range(3):
                    k = 9 * r + 3 * t
                    cand = (wbuf[k, sl] + wbuf[k + 1, sl] + wbuf[k + 2, sl]
                            + pbuf[3 * r + t, sl])
                    best = cand if best is None else jnp.maximum(best, cand)
                obuf[r, sl] = best * (SCALE / 3.0)
                return c2
            lax.fori_loop(0, HID // LANES, col, 0)
        pltpu.sync_copy(obuf, h_hbm.at[b, pl.ds(C * i, C)])
        return carry

    lax.fori_loop(0, NCHUNK, chunk, 0)


def kernel(src, W_emb, pos_emb):
    h, m = _enc(src.reshape(-1), W_emb, pos_emb)
    return h, (m != 0)[:, None, None, :]
